# Initial kernel scaffold; baseline (speedup 1.0000x reference)
import jax
import jax.numpy as jnp
from jax.experimental import pallas as pl


def kernel(x, edge_index, batch, Wrel, brel, Wroot, W_ih0, W_ih_rest, W_hh, b_ih, b_hh, W1, b1, W2, b2):
    raise NotImplementedError("write your pallas kernel here")


# speedup vs baseline: 16.5808x; 16.5808x over previous
import functools

import jax
import jax.numpy as jnp
from jax import lax
from jax.experimental import pallas as pl
from jax.experimental.pallas import tpu as pltpu
from jax.experimental.pallas import tpu_sc as plsc

N = 10000
E = 320000
D = 128
H = 8
B = 64
C = 10
L = 12
T = 12

NC = 2
NS = 16
NW = NC * NS
EPW = E // NW
CH = 128
NCHUNK = EPW // CH
TAIL = EPW - NCHUNK * CH
NPAD = 10240
RPT = NPAD // NS


def _proj_body(x_ref, wrel_ref, wroot_ref, xr_ref, xroot_ref):
    xb = x_ref[...]
    xr_ref[...] = jnp.dot(xb, wrel_ref[...], preferred_element_type=jnp.float32)
    xroot_ref[...] = jnp.dot(xb, wroot_ref[...], preferred_element_type=jnp.float32)


def _project(x, Wrel, Wroot):
    blk = 2000
    grid = N // blk
    return pl.pallas_call(
        _proj_body,
        grid=(grid,),
        in_specs=[
            pl.BlockSpec((blk, D), lambda i: (i, 0)),
            pl.BlockSpec((D, H), lambda i: (0, 0)),
            pl.BlockSpec((D, H), lambda i: (0, 0)),
        ],
        out_specs=[
            pl.BlockSpec((blk, H), lambda i: (i, 0)),
            pl.BlockSpec((blk, H), lambda i: (i, 0)),
        ],
        out_shape=[
            jax.ShapeDtypeStruct((N, H), jnp.float32),
            jax.ShapeDtypeStruct((N, H), jnp.float32),
        ],
    )(x, Wrel, Wroot)


def _sc_scatter_body(src_hbm, dst_hbm, xr_hbm, zer_hbm, out_hbm,
                     sidx, didx, rows, sidx_t, didx_t, rows_t, stage, agg_sh,
                     sem):
    cid = lax.axis_index("c")
    sid = lax.axis_index("s")

    pltpu.sync_copy(zer_hbm, stage)
    pltpu.sync_copy(stage, agg_sh.at[pl.ds(sid * RPT, RPT)])
    plsc.subcore_barrier()

    base0 = (cid * NS + sid) * EPW

    @pl.loop(0, NCHUNK)
    def _(j):
        base = base0 + j * CH
        pltpu.sync_copy(src_hbm.at[pl.ds(base, CH)], sidx)
        pltpu.sync_copy(dst_hbm.at[pl.ds(base, CH)], didx)
        pltpu.async_copy(xr_hbm.at[sidx], rows, sem).wait()
        pltpu.sync_copy(rows, agg_sh.at[didx], add=True)

    base = base0 + NCHUNK * CH
    pltpu.sync_copy(src_hbm.at[pl.ds(base, TAIL)], sidx_t)
    pltpu.sync_copy(dst_hbm.at[pl.ds(base, TAIL)], didx_t)
    pltpu.async_copy(xr_hbm.at[sidx_t], rows_t, sem).wait()
    pltpu.sync_copy(rows_t, agg_sh.at[didx_t], add=True)

    plsc.subcore_barrier()
    pltpu.sync_copy(agg_sh.at[pl.ds(sid * RPT, RPT)], stage)
    pltpu.sync_copy(stage, out_hbm.at[cid, pl.ds(sid * RPT, RPT)])


@functools.lru_cache(maxsize=1)
def _sc_scatter_kernel():
    return pl.kernel(
        _sc_scatter_body,
        out_type=jax.ShapeDtypeStruct((NC, NPAD, H), jnp.float32),
        mesh=plsc.VectorSubcoreMesh(core_axis_name="c", subcore_axis_name="s",
                                    num_cores=NC, num_subcores=NS),
        compiler_params=pltpu.CompilerParams(use_tc_tiling_on_sc=False),
        scratch_types=[
            pltpu.VMEM((CH,), jnp.int32),
            pltpu.VMEM((CH,), jnp.int32),
            pltpu.VMEM((CH, H), jnp.float32),
            pltpu.VMEM((TAIL,), jnp.int32),
            pltpu.VMEM((TAIL,), jnp.int32),
            pltpu.VMEM((TAIL, H), jnp.float32),
            pltpu.VMEM((RPT, H), jnp.float32),
            pltpu.VMEM_SHARED((NPAD, H), jnp.float32),
            pltpu.SemaphoreType.DMA,
        ],
    )


def _dot(x, y, dims):
    return lax.dot_general(x, y, (dims, ((), ())),
                           preferred_element_type=jnp.float32)


def _s2s_body(agg_ref, xroot_ref, batch_ref, brel_ref,
              wih0_ref, wihr_ref, whh_ref, bih_ref, bhh_ref,
              w1_ref, b1_ref, w2_ref, b2_ref, out_ref):
    h_node = (agg_ref[0, :N] + agg_ref[1, :N] + xroot_ref[...]
              + brel_ref[...])
    h_nodet = jnp.transpose(h_node)
    batch = batch_ref[...]
    gid = lax.broadcasted_iota(jnp.int32, (B, N), 0)
    seg = gid == batch
    segf = seg.astype(jnp.float32)

    wih0 = wih0_ref[...]
    wihr = wihr_ref[...]
    whh = whh_ref[...]
    bih = bih_ref[...]
    bhh = bhh_ref[...]

    neg_inf = jnp.float32(-jnp.inf)

    def tstep(_, carry):
        q_star, h, c = carry
        xs = q_star
        hs, cs = [], []
        for l in range(L):
            wih = wih0 if l == 0 else wihr[l - 1]
            gates = (_dot(xs, wih, ((1,), (0,)))
                     + bih[l][None, :]
                     + _dot(h[l], whh[l], ((1,), (0,)))
                     + bhh[l][None, :])
            i_g = jax.nn.sigmoid(gates[:, 0 * H:1 * H])
            f_g = jax.nn.sigmoid(gates[:, 1 * H:2 * H])
            g_g = jnp.tanh(gates[:, 2 * H:3 * H])
            o_g = jax.nn.sigmoid(gates[:, 3 * H:4 * H])
            c_l = f_g * c[l] + i_g * g_g
            h_l = o_g * jnp.tanh(c_l)
            hs.append(h_l)
            cs.append(c_l)
            xs = h_l
        q = xs

        qnt = _dot(q, segf, ((0,), (0,)))
        e = jnp.sum(h_nodet * qnt, axis=0, keepdims=True)
        em = jnp.where(seg, e, neg_inf)
        m = jnp.max(em, axis=1, keepdims=True)
        m0 = jnp.where(m == neg_inf, 0.0, m)
        mn = _dot(m0, segf, ((0,), (0,)))
        ex = jnp.exp(e - mn)
        den = _dot(ex, segf, ((1,), (1,)))
        dn = _dot(den, segf, ((1,), (0,)))
        a = ex / (dn + 1e-16)
        r = _dot(segf, a * h_nodet, ((1,), (1,)))
        q_star = jnp.concatenate([q, r], axis=1)
        return q_star, jnp.stack(hs), jnp.stack(cs)

    q_star, _, _ = lax.fori_loop(
        0, T, tstep,
        (jnp.zeros((B, 2 * H), jnp.float32),
         jnp.zeros((L, B, H), jnp.float32),
         jnp.zeros((L, B, H), jnp.float32)))

    o1 = jnp.maximum(
        jnp.dot(q_star, w1_ref[...], preferred_element_type=jnp.float32)
        + b1_ref[...], 0.0)
    o2 = (jnp.dot(o1, w2_ref[...], preferred_element_type=jnp.float32)
          + b2_ref[...])
    mx = jnp.max(o2, axis=1, keepdims=True)
    lse = mx + jnp.log(jnp.sum(jnp.exp(o2 - mx), axis=1, keepdims=True))
    out_ref[...] = o2 - lse


def _set2set(agg, xroot, batch2d, brel2d, wih0t, wihrt, whht, bih, bhh,
             W1, b12d, W2, b22d):
    return pl.pallas_call(
        _s2s_body,
        out_shape=jax.ShapeDtypeStruct((B, C), jnp.float32),
    )(agg, xroot, batch2d, brel2d, wih0t, wihrt, whht, bih, bhh,
      W1, b12d, W2, b22d)


def kernel(x, edge_index, batch, Wrel, brel, Wroot, W_ih0, W_ih_rest, W_hh,
           b_ih, b_hh, W1, b1, W2, b2):
    xr, xroot = _project(x, Wrel, Wroot)
    zer = jnp.zeros((RPT, H), jnp.float32)
    agg = _sc_scatter_kernel()(edge_index[0], edge_index[1], xr, zer)
    out = _set2set(
        agg, xroot,
        batch.reshape(1, N),
        brel.reshape(1, H),
        W_ih0.T,
        jnp.transpose(W_ih_rest, (0, 2, 1)),
        jnp.transpose(W_hh, (0, 2, 1)),
        b_ih, b_hh,
        W1, b1.reshape(1, 2 * H), W2, b2.reshape(1, C),
    )
    return out


# speedup vs baseline: 25.4951x; 1.5376x over previous
import functools

import jax
import jax.numpy as jnp
from jax import lax
from jax.experimental import pallas as pl
from jax.experimental.pallas import tpu as pltpu
from jax.experimental.pallas import tpu_sc as plsc

N = 10000
E = 320000
D = 128
H = 8
B = 64
C = 10
L = 12
T = 12

NC = 2
NS = 16
NW = NC * NS
CH = 128
EROWS = (E + NW * CH - 1) // (NW * CH) * NW
EPAD = EROWS * CH
RPW = EROWS // NW
NPAD = 10240
RPT = NPAD // NS


def _proj_body(x_ref, wrel_ref, wroot_ref, xr_ref, xroot_ref):
    xb = x_ref[...]
    xr_ref[...] = jnp.dot(xb, wrel_ref[...], preferred_element_type=jnp.float32)
    xroot_ref[...] = jnp.dot(xb, wroot_ref[...], preferred_element_type=jnp.float32)


def _project(x, Wrel, Wroot):
    blk = 2000
    grid = N // blk
    return pl.pallas_call(
        _proj_body,
        grid=(grid,),
        in_specs=[
            pl.BlockSpec((blk, D), lambda i: (i, 0)),
            pl.BlockSpec((D, H), lambda i: (0, 0)),
            pl.BlockSpec((D, H), lambda i: (0, 0)),
        ],
        out_specs=[
            pl.BlockSpec((blk, H), lambda i: (i, 0)),
            pl.BlockSpec((blk, H), lambda i: (i, 0)),
        ],
        out_shape=[
            jax.ShapeDtypeStruct((N, H), jnp.float32),
            jax.ShapeDtypeStruct((N, H), jnp.float32),
        ],
    )(x, Wrel, Wroot)


def _sc_scatter_body(src_hbm, dst_hbm, xr_hbm, zer_hbm, out_hbm,
                     sidx, didx, rows, stage, agg_sh, sem):
    cid = lax.axis_index("c")
    sid = lax.axis_index("s")

    pltpu.sync_copy(zer_hbm, stage)
    pltpu.sync_copy(stage, agg_sh.at[pl.ds(sid * RPT, RPT)])

    row0 = (cid * NS + sid) * RPW
    pltpu.sync_copy(src_hbm.at[pl.ds(row0, RPW)], sidx)
    pltpu.sync_copy(dst_hbm.at[pl.ds(row0, RPW)], didx)

    @pl.loop(0, RPW)
    def _(j):
        pltpu.async_copy(xr_hbm.at[sidx.at[j]], rows.at[j], sem)

    @pl.loop(0, RPW)
    def _(j):
        pltpu.make_async_copy(xr_hbm.at[sidx.at[j]], rows.at[j], sem).wait()

    plsc.subcore_barrier()

    @pl.loop(0, RPW)
    def _(j):
        pltpu.async_copy(rows.at[j], agg_sh.at[didx.at[j]], sem, add=True)

    @pl.loop(0, RPW)
    def _(j):
        pltpu.make_async_copy(rows.at[j], agg_sh.at[didx.at[j]], sem).wait()

    plsc.subcore_barrier()
    pltpu.sync_copy(agg_sh.at[pl.ds(sid * RPT, RPT)], stage)
    pltpu.sync_copy(stage, out_hbm.at[cid, pl.ds(sid * RPT, RPT)])


@functools.lru_cache(maxsize=1)
def _sc_scatter_kernel():
    return pl.kernel(
        _sc_scatter_body,
        out_type=jax.ShapeDtypeStruct((NC, NPAD, H), jnp.float32),
        mesh=plsc.VectorSubcoreMesh(core_axis_name="c", subcore_axis_name="s",
                                    num_cores=NC, num_subcores=NS),
        compiler_params=pltpu.CompilerParams(use_tc_tiling_on_sc=False),
        scratch_types=[
            pltpu.VMEM((RPW, CH), jnp.int32),
            pltpu.VMEM((RPW, CH), jnp.int32),
            pltpu.VMEM((RPW, CH, H), jnp.float32),
            pltpu.VMEM((RPT, H), jnp.float32),
            pltpu.VMEM_SHARED((NPAD, H), jnp.float32),
            pltpu.SemaphoreType.DMA,
        ],
    )


def _dot(x, y, dims):
    return lax.dot_general(x, y, (dims, ((), ())),
                           preferred_element_type=jnp.float32)


def _s2s_body(agg_ref, xroot_ref, batch_ref, brel_ref,
              wih0_ref, wihr_ref, whh_ref, bih_ref, bhh_ref,
              w1_ref, b1_ref, w2_ref, b2_ref, out_ref):
    h_node = (agg_ref[0, :N] + agg_ref[1, :N] + xroot_ref[...]
              + brel_ref[...])
    h_nodet = jnp.transpose(h_node)
    batch = batch_ref[...]
    gid = lax.broadcasted_iota(jnp.int32, (B, N), 0)
    seg = gid == batch
    segf = seg.astype(jnp.float32)

    wih0 = wih0_ref[...]
    wihr = wihr_ref[...]
    whh = whh_ref[...]
    bih = bih_ref[...]
    bhh = bhh_ref[...]

    neg_inf = jnp.float32(-jnp.inf)

    def tstep(_, carry):
        q_star, h, c = carry
        xs = q_star
        hs, cs = [], []
        for l in range(L):
            wih = wih0 if l == 0 else wihr[l - 1]
            gates = (_dot(xs, wih, ((1,), (0,)))
                     + bih[l][None, :]
                     + _dot(h[l], whh[l], ((1,), (0,)))
                     + bhh[l][None, :])
            i_g = jax.nn.sigmoid(gates[:, 0 * H:1 * H])
            f_g = jax.nn.sigmoid(gates[:, 1 * H:2 * H])
            g_g = jnp.tanh(gates[:, 2 * H:3 * H])
            o_g = jax.nn.sigmoid(gates[:, 3 * H:4 * H])
            c_l = f_g * c[l] + i_g * g_g
            h_l = o_g * jnp.tanh(c_l)
            hs.append(h_l)
            cs.append(c_l)
            xs = h_l
        q = xs

        qnt = _dot(q, segf, ((0,), (0,)))
        e = jnp.sum(h_nodet * qnt, axis=0, keepdims=True)
        em = jnp.where(seg, e, neg_inf)
        m = jnp.max(em, axis=1, keepdims=True)
        m0 = jnp.where(m == neg_inf, 0.0, m)
        mn = _dot(m0, segf, ((0,), (0,)))
        ex = jnp.exp(e - mn)
        den = _dot(ex, segf, ((1,), (1,)))
        dn = _dot(den, segf, ((1,), (0,)))
        a = ex / (dn + 1e-16)
        r = _dot(segf, a * h_nodet, ((1,), (1,)))
        q_star = jnp.concatenate([q, r], axis=1)
        return q_star, jnp.stack(hs), jnp.stack(cs)

    q_star, _, _ = lax.fori_loop(
        0, T, tstep,
        (jnp.zeros((B, 2 * H), jnp.float32),
         jnp.zeros((L, B, H), jnp.float32),
         jnp.zeros((L, B, H), jnp.float32)))

    o1 = jnp.maximum(
        jnp.dot(q_star, w1_ref[...], preferred_element_type=jnp.float32)
        + b1_ref[...], 0.0)
    o2 = (jnp.dot(o1, w2_ref[...], preferred_element_type=jnp.float32)
          + b2_ref[...])
    mx = jnp.max(o2, axis=1, keepdims=True)
    lse = mx + jnp.log(jnp.sum(jnp.exp(o2 - mx), axis=1, keepdims=True))
    out_ref[...] = o2 - lse


def _set2set(agg, xroot, batch2d, brel2d, wih0t, wihrt, whht, bih, bhh,
             W1, b12d, W2, b22d):
    return pl.pallas_call(
        _s2s_body,
        out_shape=jax.ShapeDtypeStruct((B, C), jnp.float32),
    )(agg, xroot, batch2d, brel2d, wih0t, wihrt, whht, bih, bhh,
      W1, b12d, W2, b22d)


def kernel(x, edge_index, batch, Wrel, brel, Wroot, W_ih0, W_ih_rest, W_hh,
           b_ih, b_hh, W1, b1, W2, b2):
    xr, xroot = _project(x, Wrel, Wroot)
    zer = jnp.zeros((RPT, H), jnp.float32)
    npd = EPAD - E
    src2 = jnp.concatenate(
        [edge_index[0], jnp.zeros((npd,), jnp.int32)]).reshape(EROWS, CH)
    dst2 = jnp.concatenate(
        [edge_index[1],
         N + jnp.arange(npd, dtype=jnp.int32) % (NPAD - N)]).reshape(EROWS, CH)
    agg = _sc_scatter_kernel()(src2, dst2, xr, zer)
    out = _set2set(
        agg, xroot,
        batch.reshape(1, N),
        brel.reshape(1, H),
        W_ih0.T,
        jnp.transpose(W_ih_rest, (0, 2, 1)),
        jnp.transpose(W_hh, (0, 2, 1)),
        b_ih, b_hh,
        W1, b1.reshape(1, 2 * H), W2, b2.reshape(1, C),
    )
    return out


# speedup vs baseline: 28.6534x; 1.1239x over previous
import functools

import jax
import jax.numpy as jnp
from jax import lax
from jax.experimental import pallas as pl
from jax.experimental.pallas import tpu as pltpu
from jax.experimental.pallas import tpu_sc as plsc

N = 10000
E = 320000
D = 128
H = 8
B = 64
C = 10
L = 12
T = 12

NC = 2
NS = 16
NW = NC * NS
CH = 128
EROWS = (E + NW * CH - 1) // (NW * CH) * NW
EPAD = EROWS * CH
RPW = EROWS // NW
NPAD = 10240
RPT = NPAD // NS


def _proj_body(x_ref, wrel_ref, wroott_ref, xr_ref, xroott_ref):
    xb = x_ref[...]
    xr_ref[...] = jnp.dot(xb, wrel_ref[...], preferred_element_type=jnp.float32)
    xroott_ref[...] = lax.dot_general(
        wroott_ref[...], xb, (((1,), (1,)), ((), ())),
        preferred_element_type=jnp.float32)


def _project(x, Wrel, Wroot):
    return pl.pallas_call(
        _proj_body,
        out_shape=[
            jax.ShapeDtypeStruct((N, H), jnp.float32),
            jax.ShapeDtypeStruct((H, N), jnp.float32),
        ],
    )(x, Wrel, Wroot.T)


def _sc_scatter_body(src_hbm, dst_hbm, xr_hbm, zer_hbm, out_hbm,
                     sidx, didx, rows, stage, agg_sh, sem):
    cid = lax.axis_index("c")
    sid = lax.axis_index("s")

    pltpu.sync_copy(zer_hbm, stage)
    pltpu.sync_copy(stage, agg_sh.at[pl.ds(sid * RPT, RPT)])

    row0 = (cid * NS + sid) * RPW
    pltpu.sync_copy(src_hbm.at[pl.ds(row0, RPW)], sidx)
    pltpu.sync_copy(dst_hbm.at[pl.ds(row0, RPW)], didx)

    @pl.loop(0, RPW)
    def _(j):
        pltpu.async_copy(xr_hbm.at[sidx.at[j]], rows.at[j], sem)

    @pl.loop(0, RPW)
    def _(j):
        pltpu.make_async_copy(xr_hbm.at[sidx.at[j]], rows.at[j], sem).wait()

    plsc.subcore_barrier()

    @pl.loop(0, RPW)
    def _(j):
        pltpu.async_copy(rows.at[j], agg_sh.at[didx.at[j]], sem, add=True)

    @pl.loop(0, RPW)
    def _(j):
        pltpu.make_async_copy(rows.at[j], agg_sh.at[didx.at[j]], sem).wait()

    plsc.subcore_barrier()
    pltpu.sync_copy(agg_sh.at[pl.ds(sid * RPT, RPT)], stage)
    pltpu.sync_copy(stage, out_hbm.at[cid, pl.ds(sid * RPT, RPT)])


@functools.lru_cache(maxsize=1)
def _sc_scatter_kernel():
    return pl.kernel(
        _sc_scatter_body,
        out_type=jax.ShapeDtypeStruct((NC, NPAD, H), jnp.float32),
        mesh=plsc.VectorSubcoreMesh(core_axis_name="c", subcore_axis_name="s",
                                    num_cores=NC, num_subcores=NS),
        compiler_params=pltpu.CompilerParams(use_tc_tiling_on_sc=False),
        scratch_types=[
            pltpu.VMEM((RPW, CH), jnp.int32),
            pltpu.VMEM((RPW, CH), jnp.int32),
            pltpu.VMEM((RPW, CH, H), jnp.float32),
            pltpu.VMEM((RPT, H), jnp.float32),
            pltpu.VMEM_SHARED((NPAD, H), jnp.float32),
            pltpu.SemaphoreType.DMA,
        ],
    )


def _dot(x, y, dims):
    return lax.dot_general(x, y, (dims, ((), ())),
                           preferred_element_type=jnp.float32)


def _s2s_body(agg_ref, xroott_ref, batch_ref, brel_ref,
              wcat0_ref, wcatr_ref, bsum_ref,
              w1_ref, b1_ref, w2_ref, b2_ref, out_ref):
    NP = NPAD
    sum2 = agg_ref[0] + agg_ref[1]
    xroott = jnp.concatenate(
        [xroott_ref[...] + brel_ref[...],
         jnp.zeros((H, NP - N), jnp.float32)], axis=1)
    h_nodet = jnp.transpose(sum2) + xroott
    haug = jnp.concatenate(
        [h_nodet, jnp.ones((1, NP), jnp.float32)], axis=0)
    batch = jnp.concatenate(
        [batch_ref[...], jnp.full((1, NP - N), B, jnp.int32)], axis=1)
    gid = lax.broadcasted_iota(jnp.int32, (B, NP), 0)
    seg = gid == batch
    segf = seg.astype(jnp.float32)
    segft = jnp.transpose(segf)
    valid = (lax.broadcasted_iota(jnp.int32, (1, NP), 1)
             < N).astype(jnp.float32)

    wcat0 = wcat0_ref[...]
    wcatr = wcatr_ref[...]
    bsum = bsum_ref[...]

    neg_inf = jnp.float32(-jnp.inf)

    def tstep(_, carry):
        q_star, h, c = carry
        xs = q_star
        hs, cs = [], []
        for l in range(L):
            wcat = wcat0 if l == 0 else wcatr[l - 1]
            inp = jnp.concatenate([xs, h[l]], axis=1)
            gates = _dot(inp, wcat, ((1,), (0,))) + bsum[l][None, :]
            i_g = jax.nn.sigmoid(gates[:, 0 * H:1 * H])
            f_g = jax.nn.sigmoid(gates[:, 1 * H:2 * H])
            g_g = jnp.tanh(gates[:, 2 * H:3 * H])
            o_g = jax.nn.sigmoid(gates[:, 3 * H:4 * H])
            c_l = f_g * c[l] + i_g * g_g
            h_l = o_g * jnp.tanh(c_l)
            hs.append(h_l)
            cs.append(c_l)
            xs = h_l
        q = xs

        qt = jnp.transpose(q)
        qnt = _dot(qt, segf, ((1,), (0,)))
        e = jnp.sum(h_nodet * qnt, axis=0, keepdims=True)
        em = jnp.where(seg, e, neg_inf)
        m = jnp.max(em, axis=1, keepdims=True)
        m0t = jnp.transpose(jnp.where(m == neg_inf, 0.0, m))
        mn = _dot(m0t, segf, ((1,), (0,)))
        ex = jnp.exp(e - mn) * valid
        rden = _dot(ex * haug, segft, ((1,), (0,)))
        r = jnp.transpose(rden[:H] / (rden[H:] + 1e-16))
        q_star = jnp.concatenate([q, r], axis=1)
        return q_star, tuple(hs), tuple(cs)

    q_star, _, _ = lax.fori_loop(
        0, T, tstep,
        (jnp.zeros((B, 2 * H), jnp.float32),
         tuple(jnp.zeros((B, H), jnp.float32) for _ in range(L)),
         tuple(jnp.zeros((B, H), jnp.float32) for _ in range(L))))

    o1 = jnp.maximum(
        jnp.dot(q_star, w1_ref[...], preferred_element_type=jnp.float32)
        + b1_ref[...], 0.0)
    o2 = (jnp.dot(o1, w2_ref[...], preferred_element_type=jnp.float32)
          + b2_ref[...])
    mx = jnp.max(o2, axis=1, keepdims=True)
    lse = mx + jnp.log(jnp.sum(jnp.exp(o2 - mx), axis=1, keepdims=True))
    out_ref[...] = o2 - lse


def _set2set(agg, xroot, batch2d, brel2d, wcat0, wcatr, bsum,
             W1, b12d, W2, b22d):
    return pl.pallas_call(
        _s2s_body,
        out_shape=jax.ShapeDtypeStruct((B, C), jnp.float32),
    )(agg, xroot, batch2d, brel2d, wcat0, wcatr, bsum,
      W1, b12d, W2, b22d)


def kernel(x, edge_index, batch, Wrel, brel, Wroot, W_ih0, W_ih_rest, W_hh,
           b_ih, b_hh, W1, b1, W2, b2):
    xr, xroott = _project(x, Wrel, Wroot)
    zer = jnp.zeros((RPT, H), jnp.float32)
    npd = EPAD - E
    src2 = jnp.concatenate(
        [edge_index[0], jnp.zeros((npd,), jnp.int32)]).reshape(EROWS, CH)
    dst2 = jnp.concatenate(
        [edge_index[1],
         N + jnp.arange(npd, dtype=jnp.int32) % (NPAD - N)]).reshape(EROWS, CH)
    agg = _sc_scatter_kernel()(src2, dst2, xr, zer)
    wcat0 = jnp.concatenate([W_ih0.T, W_hh[0].T], axis=0)
    wcatr = jnp.concatenate(
        [jnp.transpose(W_ih_rest, (0, 2, 1)),
         jnp.transpose(W_hh[1:], (0, 2, 1))], axis=1)
    out = _set2set(
        agg, xroott,
        batch.reshape(1, N),
        brel.reshape(H, 1),
        wcat0, wcatr, b_ih + b_hh,
        W1, b1.reshape(1, 2 * H), W2, b2.reshape(1, C),
    )
    return out


# speedup vs baseline: 31.0531x; 1.0837x over previous
import functools

import jax
import jax.numpy as jnp
from jax import lax
from jax.experimental import pallas as pl
from jax.experimental.pallas import tpu as pltpu
from jax.experimental.pallas import tpu_sc as plsc

N = 10000
E = 320000
D = 128
H = 8
B = 64
C = 10
L = 12
T = 12

NC = 2
NS = 16
NW = NC * NS
CH = 128
EROWS = (E + NW * CH - 1) // (NW * CH) * NW
EPAD = EROWS * CH
RPW = EROWS // NW
NPAD = 10240
RPT = NPAD // NS


def _proj_body(x_ref, wrel_ref, wroott_ref, xr_ref, xroott_ref):
    xb = x_ref[...]
    xr_ref[...] = jnp.dot(xb, wrel_ref[...], preferred_element_type=jnp.float32)
    xroott_ref[...] = lax.dot_general(
        wroott_ref[...], xb, (((1,), (1,)), ((), ())),
        preferred_element_type=jnp.float32)


def _project(x, Wrel, Wroot):
    return pl.pallas_call(
        _proj_body,
        out_shape=[
            jax.ShapeDtypeStruct((N, H), jnp.float32),
            jax.ShapeDtypeStruct((H, N), jnp.float32),
        ],
    )(x, Wrel, Wroot.T)


def _sc_scatter_body(src_hbm, dst_hbm, xr_hbm, zer_hbm, out_hbm,
                     sidx, didx, rows, stage, agg_sh, sem):
    cid = lax.axis_index("c")
    sid = lax.axis_index("s")

    pltpu.sync_copy(zer_hbm, stage)
    pltpu.sync_copy(stage, agg_sh.at[pl.ds(sid * RPT, RPT)])

    row0 = (cid * NS + sid) * RPW
    pltpu.sync_copy(src_hbm.at[pl.ds(row0, RPW)], sidx)
    pltpu.sync_copy(dst_hbm.at[pl.ds(row0, RPW)], didx)

    @pl.loop(0, RPW)
    def _(j):
        pltpu.async_copy(xr_hbm.at[sidx.at[j]], rows.at[j], sem)

    @pl.loop(0, RPW)
    def _(j):
        pltpu.make_async_copy(xr_hbm.at[sidx.at[j]], rows.at[j], sem).wait()

    plsc.subcore_barrier()

    @pl.loop(0, RPW)
    def _(j):
        pltpu.async_copy(rows.at[j], agg_sh.at[didx.at[j]], sem, add=True)

    @pl.loop(0, RPW)
    def _(j):
        pltpu.make_async_copy(rows.at[j], agg_sh.at[didx.at[j]], sem).wait()

    plsc.subcore_barrier()
    pltpu.sync_copy(agg_sh.at[pl.ds(sid * RPT, RPT)], stage)
    pltpu.sync_copy(stage, out_hbm.at[cid, pl.ds(sid * RPT, RPT)])


@functools.lru_cache(maxsize=1)
def _sc_scatter_kernel():
    return pl.kernel(
        _sc_scatter_body,
        out_type=jax.ShapeDtypeStruct((NC, NPAD, H), jnp.float32),
        mesh=plsc.VectorSubcoreMesh(core_axis_name="c", subcore_axis_name="s",
                                    num_cores=NC, num_subcores=NS),
        compiler_params=pltpu.CompilerParams(use_tc_tiling_on_sc=False),
        scratch_types=[
            pltpu.VMEM((RPW, CH), jnp.int32),
            pltpu.VMEM((RPW, CH), jnp.int32),
            pltpu.VMEM((RPW, CH, H), jnp.float32),
            pltpu.VMEM((RPT, H), jnp.float32),
            pltpu.VMEM_SHARED((NPAD, H), jnp.float32),
            pltpu.SemaphoreType.DMA,
        ],
    )


def _dot(x, y, dims):
    return lax.dot_general(x, y, (dims, ((), ())),
                           preferred_element_type=jnp.float32)


def _s2s_body(agg_ref, xroott_ref, batch_ref, brel_ref,
              wcat0_ref, wcatr_ref, bsum_ref,
              w1_ref, b1_ref, w2_ref, b2_ref, out_ref):
    NP = NPAD
    sum2 = agg_ref[0] + agg_ref[1]
    xroott = jnp.concatenate(
        [xroott_ref[...] + brel_ref[...],
         jnp.zeros((H, NP - N), jnp.float32)], axis=1)
    h_nodet = jnp.transpose(sum2) + xroott
    haug = jnp.concatenate(
        [h_nodet, jnp.ones((1, NP), jnp.float32)], axis=0)
    batch = jnp.concatenate(
        [batch_ref[...], jnp.full((1, NP - N), B, jnp.int32)], axis=1)
    gid = lax.broadcasted_iota(jnp.int32, (B, NP), 0)
    seg = gid == batch
    segf = seg.astype(jnp.float32)
    segft = jnp.transpose(segf)
    valid = (lax.broadcasted_iota(jnp.int32, (1, NP), 1)
             < N).astype(jnp.float32)

    wcat0 = wcat0_ref[...]
    wcatr = wcatr_ref[...]
    bsum = bsum_ref[...]

    neg_inf = jnp.float32(-jnp.inf)

    def tstep(_, carry):
        q_star, h, c = carry
        xs = q_star
        hs, cs = [], []
        for l in range(L):
            wcat = wcat0 if l == 0 else wcatr[l - 1]
            inp = jnp.concatenate([xs, h[l]], axis=1)
            gates = _dot(inp, wcat, ((1,), (0,))) + bsum[l][None, :]
            i_g = jax.nn.sigmoid(gates[:, 0 * H:1 * H])
            f_g = jax.nn.sigmoid(gates[:, 1 * H:2 * H])
            g_g = jnp.tanh(gates[:, 2 * H:3 * H])
            o_g = jax.nn.sigmoid(gates[:, 3 * H:4 * H])
            c_l = f_g * c[l] + i_g * g_g
            h_l = o_g * jnp.tanh(c_l)
            hs.append(h_l)
            cs.append(c_l)
            xs = h_l
        q = xs

        qt = jnp.transpose(q)
        qnt = _dot(qt, segf, ((1,), (0,)))
        e = jnp.sum(h_nodet * qnt, axis=0, keepdims=True)
        em = jnp.where(seg, e, neg_inf)
        m = jnp.max(em, axis=1, keepdims=True)
        m0t = jnp.transpose(jnp.where(m == neg_inf, 0.0, m))
        mn = _dot(m0t, segf, ((1,), (0,)))
        ex = jnp.exp(e - mn) * valid
        rden = _dot(ex * haug, segft, ((1,), (0,)))
        r = jnp.transpose(rden[:H] / (rden[H:] + 1e-16))
        q_star = jnp.concatenate([q, r], axis=1)
        return q_star, tuple(hs), tuple(cs)

    q_star, _, _ = lax.fori_loop(
        0, T, tstep,
        (jnp.zeros((B, 2 * H), jnp.float32),
         tuple(jnp.zeros((B, H), jnp.float32) for _ in range(L)),
         tuple(jnp.zeros((B, H), jnp.float32) for _ in range(L))))

    o1 = jnp.maximum(
        jnp.dot(q_star, w1_ref[...], preferred_element_type=jnp.float32)
        + b1_ref[...], 0.0)
    o2 = (jnp.dot(o1, w2_ref[...], preferred_element_type=jnp.float32)
          + b2_ref[...])
    mx = jnp.max(o2, axis=1, keepdims=True)
    lse = mx + jnp.log(jnp.sum(jnp.exp(o2 - mx), axis=1, keepdims=True))
    out_ref[...] = o2 - lse


def _set2set(agg, xroot, batch2d, brel2d, wcat0, wcatr, bsum,
             W1, b12d, W2, b22d):
    return pl.pallas_call(
        _s2s_body,
        out_shape=jax.ShapeDtypeStruct((B, C), jnp.float32),
    )(agg, xroot, batch2d, brel2d, wcat0, wcatr, bsum,
      W1, b12d, W2, b22d)


def kernel(x, edge_index, batch, Wrel, brel, Wroot, W_ih0, W_ih_rest, W_hh,
           b_ih, b_hh, W1, b1, W2, b2):
    xr, xroott = _project(x, Wrel, Wroot)
    zer = jnp.zeros((RPT, H), jnp.float32)
    npd = EPAD - E
    src2 = jnp.concatenate(
        [edge_index[0],
         jnp.arange(npd, dtype=jnp.int32) % N]).reshape(EROWS, CH)
    dst2 = jnp.concatenate(
        [edge_index[1],
         N + jnp.arange(npd, dtype=jnp.int32) % (NPAD - N)]).reshape(EROWS, CH)
    agg = _sc_scatter_kernel()(src2, dst2, xr, zer)
    wcat0 = jnp.concatenate([W_ih0.T, W_hh[0].T], axis=0)
    wcatr = jnp.concatenate(
        [jnp.transpose(W_ih_rest, (0, 2, 1)),
         jnp.transpose(W_hh[1:], (0, 2, 1))], axis=1)
    out = _set2set(
        agg, xroott,
        batch.reshape(1, N),
        brel.reshape(H, 1),
        wcat0, wcatr, b_ih + b_hh,
        W1, b1.reshape(1, 2 * H), W2, b2.reshape(1, C),
    )
    return out
